# Initial kernel scaffold; baseline (speedup 1.0000x reference)
import jax
import jax.numpy as jnp
from jax.experimental import pallas as pl


def kernel(flat, cu_seqlens, lang_proj):
    raise NotImplementedError("write your pallas kernel here")


# speedup vs baseline: 1.5548x; 1.5548x over previous
import functools

import jax
import jax.numpy as jnp
from jax import lax
from jax.experimental import pallas as pl
from jax.experimental.pallas import tpu as pltpu
from jax.experimental.pallas import tpu_sc as plsc

B = 16
TOTAL = 32768
D = 768
DP = 512
LANE = 16
NCOL = D // LANE

NC = 2
NS = 16
NW = NC * NS
RPW = TOTAL // NW
CH = 128
K = RPW // CH


def _sc_body(flat_hbm, cu2_hbm, z_hbm, out_hbm, buf, cu_v, acc_v):
    cid = lax.axis_index("c")
    sid = lax.axis_index("s")
    w = cid * NS + sid
    base = w * RPW

    pltpu.sync_copy(cu2_hbm, cu_v)
    pltpu.sync_copy(z_hbm, acc_v)

    starts_vec = cu_v[pl.ds(0, LANE)]
    ends_vec = cu_v[pl.ds(LANE, LANE)]
    starts = [starts_vec[s] for s in range(B)]
    ends = [ends_vec[s] for s in range(B)]

    def chunk_step(j, _):
        cbase = base + j * CH
        pltpu.sync_copy(flat_hbm.at[pl.ds(cbase, CH)], buf)

        for s in range(B):
            lo = jnp.clip(starts[s] - cbase, 0, CH)
            hi = jnp.clip(ends[s] - cbase, 0, CH)

            def row_step(r, _, s=s):
                for c in range(NCOL):
                    plsc.addupdate(acc_v.at[s, pl.ds(c * LANE, LANE)],
                                   buf[r, pl.ds(c * LANE, LANE)])
                return 0

            lax.fori_loop(lo, hi, row_step, 0)
        return 0

    lax.fori_loop(0, K, chunk_step, 0)

    pltpu.sync_copy(acc_v, out_hbm.at[cid, sid])


@functools.cache
def _sc_segsum():
    mesh = plsc.VectorSubcoreMesh(core_axis_name="c", subcore_axis_name="s")
    return pl.kernel(
        _sc_body,
        mesh=mesh,
        out_type=jax.ShapeDtypeStruct((NC, NS, B, D), jnp.float32),
        scratch_types=[
            pltpu.VMEM((CH, D), jnp.float32),
            pltpu.VMEM((2 * LANE,), jnp.int32),
            pltpu.VMEM((B, D), jnp.float32),
        ],
    )


def _finish_body(part_ref, cu_lo_ref, cu_hi_ref, proj_ref, out_ref):
    part = part_ref[...]
    pooled = jnp.sum(part.reshape(NC * NS, B, D), axis=0)
    cnt = (cu_hi_ref[...] - cu_lo_ref[...]).astype(jnp.float32)
    pooled = pooled / jnp.maximum(cnt, 1.0)
    ce = jnp.dot(pooled, proj_ref[...], preferred_element_type=jnp.float32)
    n = jnp.sqrt(jnp.sum(ce * ce, axis=1, keepdims=True))
    out_ref[...] = ce / (n + 1e-7)


_finish = pl.pallas_call(
    _finish_body,
    out_shape=jax.ShapeDtypeStruct((B, DP), jnp.float32),
)


def kernel(flat, cu_seqlens, lang_proj):
    cu = cu_seqlens.astype(jnp.int32)
    cu2 = jnp.concatenate([cu[0:B], cu[1:B + 1]])
    zeros = jnp.zeros((B, D), jnp.float32)
    partial = _sc_segsum()(flat, cu2, zeros)
    cu_lo = cu[:-1].reshape(B, 1)
    cu_hi = cu[1:].reshape(B, 1)
    return _finish(partial, cu_lo, cu_hi, lang_proj)


# speedup vs baseline: 6.2186x; 3.9996x over previous
import functools

import jax
import jax.numpy as jnp
from jax import lax
from jax.experimental import pallas as pl
from jax.experimental.pallas import tpu as pltpu
from jax.experimental.pallas import tpu_sc as plsc

B = 16
TOTAL = 32768
D = 768
DP = 512
LANE = 16
NCOL = D // LANE

NC = 2
NS = 16
NW = NC * NS
SC_ROWS = 16384
TC_ROWS = TOTAL - SC_ROWS
RPW = SC_ROWS // NW
CH = 64
K = RPW // CH
BLK = 1024


def _sc_body(flat_hbm, cu2_hbm, z_hbm, out_hbm, buf, cu_v, acc_v, sem):
    cid = lax.axis_index("c")
    sid = lax.axis_index("s")
    w = cid * NS + sid
    base = w * RPW

    pltpu.sync_copy(cu2_hbm, cu_v)
    pltpu.sync_copy(z_hbm, acc_v)

    starts_vec = cu_v[pl.ds(0, LANE)]
    ends_vec = cu_v[pl.ds(LANE, LANE)]
    starts = [starts_vec[s] for s in range(B)]
    ends = [ends_vec[s] for s in range(B)]

    def dma(j, pb):
        return pltpu.make_async_copy(
            flat_hbm.at[pl.ds(base + j * CH, CH)], buf.at[pb], sem.at[pb])

    dma(0, 0).start()

    zero = jnp.zeros((LANE,), jnp.float32)

    def chunk_step(j, _):
        cbase = base + j * CH
        pb = lax.rem(j, 2)

        @pl.when(j + 1 < K)
        def _():
            dma(j + 1, 1 - pb).start()

        dma(j, pb).wait()

        def row_step_range(lo, hi):
            def row_step(r, accs, pb=pb):
                return tuple(a + buf[pb, r, pl.ds(c * LANE, LANE)]
                             for c, a in enumerate(accs))
            return lax.fori_loop(lo, hi, row_step,
                                 tuple(zero for _ in range(NCOL)))

        one = jnp.int32(1)
        nil = jnp.int32(0)
        segf = nil
        segl = nil
        for s in range(B):
            segf = segf + jnp.where(ends[s] <= cbase, one, nil)
            segl = segl + jnp.where(ends[s] <= cbase + (CH - 1), one, nil)
        crosses = segf != segl

        @pl.when(jnp.logical_not(crosses))
        def _fast():
            accs = row_step_range(0, CH)
            for c in range(NCOL):
                plsc.addupdate(acc_v.at[segf, pl.ds(c * LANE, LANE)],
                               accs[c])

        @pl.when(crosses)
        def _slow():
            for s in range(B):
                lo = jnp.clip(starts[s] - cbase, 0, CH)
                hi = jnp.clip(ends[s] - cbase, 0, CH)
                accs = row_step_range(lo, hi)

                @pl.when(hi > lo)
                def _(s=s, accs=accs):
                    for c in range(NCOL):
                        plsc.addupdate(
                            acc_v.at[s, pl.ds(c * LANE, LANE)],
                            accs[c])
        return 0

    lax.fori_loop(0, K, chunk_step, 0)

    pltpu.sync_copy(acc_v, out_hbm.at[cid, sid])


@functools.cache
def _sc_segsum():
    mesh = plsc.VectorSubcoreMesh(core_axis_name="c", subcore_axis_name="s")
    return pl.kernel(
        _sc_body,
        mesh=mesh,
        out_type=jax.ShapeDtypeStruct((NC, NS, B, D), jnp.float32),
        scratch_types=[
            pltpu.VMEM((2, CH, D), jnp.float32),
            pltpu.VMEM((2 * LANE,), jnp.int32),
            pltpu.VMEM((B, D), jnp.float32),
            pltpu.SemaphoreType.DMA((2,)),
        ],
    )


def _tc_segsum_body(flat_ref, cu_lo_ref, cu_hi_ref, out_ref):
    i = pl.program_id(0)
    rows = (SC_ROWS + i * BLK
            + lax.broadcasted_iota(jnp.int32, (1, BLK), 1))
    oh = ((rows >= cu_lo_ref[...]) & (rows < cu_hi_ref[...]))
    partial = jnp.dot(oh.astype(jnp.bfloat16),
                      flat_ref[...].astype(jnp.bfloat16),
                      preferred_element_type=jnp.float32)

    @pl.when(i == 0)
    def _():
        out_ref[...] = partial

    @pl.when(i != 0)
    def _():
        out_ref[...] = out_ref[...] + partial


_tc_segsum = pl.pallas_call(
    _tc_segsum_body,
    grid=(TC_ROWS // BLK,),
    in_specs=[
        pl.BlockSpec((BLK, D), lambda i: (SC_ROWS // BLK + i, 0)),
        pl.BlockSpec((B, 1), lambda i: (0, 0)),
        pl.BlockSpec((B, 1), lambda i: (0, 0)),
    ],
    out_specs=pl.BlockSpec((B, D), lambda i: (0, 0)),
    out_shape=jax.ShapeDtypeStruct((B, D), jnp.float32),
)


def _finish_body(part_ref, tcpart_ref, cu_lo_ref, cu_hi_ref, proj_ref,
                 out_ref):
    part = part_ref[...]
    pooled = jnp.sum(part.reshape(NC * NS, B, D), axis=0) + tcpart_ref[...]
    cnt = (cu_hi_ref[...] - cu_lo_ref[...]).astype(jnp.float32)
    pooled = pooled / jnp.maximum(cnt, 1.0)
    ce = jnp.dot(pooled, proj_ref[...], preferred_element_type=jnp.float32)
    n = jnp.sqrt(jnp.sum(ce * ce, axis=1, keepdims=True))
    out_ref[...] = ce / (n + 1e-7)


_finish = pl.pallas_call(
    _finish_body,
    out_shape=jax.ShapeDtypeStruct((B, DP), jnp.float32),
)


def kernel(flat, cu_seqlens, lang_proj):
    cu = cu_seqlens.astype(jnp.int32)
    cu2 = jnp.concatenate([cu[0:B], cu[1:B + 1]])
    zeros = jnp.zeros((B, D), jnp.float32)
    cu_lo = cu[:-1].reshape(B, 1)
    cu_hi = cu[1:].reshape(B, 1)
    partial = _sc_segsum()(flat, cu2, zeros)
    tcpart = _tc_segsum(flat, cu_lo, cu_hi)
    return _finish(partial, tcpart, cu_lo, cu_hi, lang_proj)


# speedup vs baseline: 6.2514x; 1.0053x over previous
import functools

import jax
import jax.numpy as jnp
from jax import lax
from jax.experimental import pallas as pl
from jax.experimental.pallas import tpu as pltpu
from jax.experimental.pallas import tpu_sc as plsc

B = 16
TOTAL = 32768
D = 768
DP = 512
LANE = 16
NCOL = D // LANE

NC = 2
NS = 16
NW = NC * NS
SC_ROWS = 14336
TC_ROWS = TOTAL - SC_ROWS
RPW = SC_ROWS // NW
CH = 64
K = RPW // CH
BLK = 1024


def _sc_body(flat_hbm, cu2_hbm, z_hbm, out_hbm, buf, cu_v, acc_v, sem):
    cid = lax.axis_index("c")
    sid = lax.axis_index("s")
    w = cid * NS + sid
    base = w * RPW

    pltpu.sync_copy(cu2_hbm, cu_v)
    pltpu.sync_copy(z_hbm, acc_v)

    starts_vec = cu_v[pl.ds(0, LANE)]
    ends_vec = cu_v[pl.ds(LANE, LANE)]
    starts = [starts_vec[s] for s in range(B)]
    ends = [ends_vec[s] for s in range(B)]

    def dma(j, pb):
        return pltpu.make_async_copy(
            flat_hbm.at[pl.ds(base + j * CH, CH)], buf.at[pb], sem.at[pb])

    dma(0, 0).start()

    zero = jnp.zeros((LANE,), jnp.float32)

    def chunk_step(j, _):
        cbase = base + j * CH
        pb = lax.rem(j, 2)

        @pl.when(j + 1 < K)
        def _():
            dma(j + 1, 1 - pb).start()

        dma(j, pb).wait()

        def row_step_range(lo, hi):
            def row_step(r, accs, pb=pb):
                return tuple(a + buf[pb, r, pl.ds(c * LANE, LANE)]
                             for c, a in enumerate(accs))
            return lax.fori_loop(lo, hi, row_step,
                                 tuple(zero for _ in range(NCOL)))

        one = jnp.int32(1)
        nil = jnp.int32(0)
        segf = nil
        segl = nil
        for s in range(B):
            segf = segf + jnp.where(ends[s] <= cbase, one, nil)
            segl = segl + jnp.where(ends[s] <= cbase + (CH - 1), one, nil)
        crosses = segf != segl

        @pl.when(jnp.logical_not(crosses))
        def _fast():
            accs = row_step_range(0, CH)
            for c in range(NCOL):
                plsc.addupdate(acc_v.at[segf, pl.ds(c * LANE, LANE)],
                               accs[c])

        @pl.when(crosses)
        def _slow():
            for s in range(B):
                lo = jnp.clip(starts[s] - cbase, 0, CH)
                hi = jnp.clip(ends[s] - cbase, 0, CH)
                accs = row_step_range(lo, hi)

                @pl.when(hi > lo)
                def _(s=s, accs=accs):
                    for c in range(NCOL):
                        plsc.addupdate(
                            acc_v.at[s, pl.ds(c * LANE, LANE)],
                            accs[c])
        return 0

    lax.fori_loop(0, K, chunk_step, 0)

    pltpu.sync_copy(acc_v, out_hbm.at[cid, sid])


@functools.cache
def _sc_segsum():
    mesh = plsc.VectorSubcoreMesh(core_axis_name="c", subcore_axis_name="s")
    return pl.kernel(
        _sc_body,
        mesh=mesh,
        out_type=jax.ShapeDtypeStruct((NC, NS, B, D), jnp.float32),
        scratch_types=[
            pltpu.VMEM((2, CH, D), jnp.float32),
            pltpu.VMEM((2 * LANE,), jnp.int32),
            pltpu.VMEM((B, D), jnp.float32),
            pltpu.SemaphoreType.DMA((2,)),
        ],
    )


def _tc_segsum_body(flat_ref, cu_lo_ref, cu_hi_ref, out_ref):
    i = pl.program_id(0)
    rows = (SC_ROWS + i * BLK
            + lax.broadcasted_iota(jnp.int32, (1, BLK), 1))
    oh = ((rows >= cu_lo_ref[...]) & (rows < cu_hi_ref[...]))
    partial = jnp.dot(oh.astype(jnp.bfloat16),
                      flat_ref[...].astype(jnp.bfloat16),
                      preferred_element_type=jnp.float32)

    @pl.when(i == 0)
    def _():
        out_ref[...] = partial

    @pl.when(i != 0)
    def _():
        out_ref[...] = out_ref[...] + partial


_tc_segsum = pl.pallas_call(
    _tc_segsum_body,
    grid=(TC_ROWS // BLK,),
    in_specs=[
        pl.BlockSpec((BLK, D), lambda i: (SC_ROWS // BLK + i, 0)),
        pl.BlockSpec((B, 1), lambda i: (0, 0)),
        pl.BlockSpec((B, 1), lambda i: (0, 0)),
    ],
    out_specs=pl.BlockSpec((B, D), lambda i: (0, 0)),
    out_shape=jax.ShapeDtypeStruct((B, D), jnp.float32),
)


def _finish_body(part_ref, tcpart_ref, cu_lo_ref, cu_hi_ref, proj_ref,
                 out_ref):
    part = part_ref[...]
    pooled = jnp.sum(part.reshape(NC * NS, B, D), axis=0) + tcpart_ref[...]
    cnt = (cu_hi_ref[...] - cu_lo_ref[...]).astype(jnp.float32)
    pooled = pooled / jnp.maximum(cnt, 1.0)
    ce = jnp.dot(pooled, proj_ref[...], preferred_element_type=jnp.float32)
    n = jnp.sqrt(jnp.sum(ce * ce, axis=1, keepdims=True))
    out_ref[...] = ce / (n + 1e-7)


_finish = pl.pallas_call(
    _finish_body,
    out_shape=jax.ShapeDtypeStruct((B, DP), jnp.float32),
)


def kernel(flat, cu_seqlens, lang_proj):
    cu = cu_seqlens.astype(jnp.int32)
    cu2 = jnp.concatenate([cu[0:B], cu[1:B + 1]])
    zeros = jnp.zeros((B, D), jnp.float32)
    cu_lo = cu[:-1].reshape(B, 1)
    cu_hi = cu[1:].reshape(B, 1)
    partial = _sc_segsum()(flat, cu2, zeros)
    tcpart = _tc_segsum(flat, cu_lo, cu_hi)
    return _finish(partial, tcpart, cu_lo, cu_hi, lang_proj)


# speedup vs baseline: 6.5938x; 1.0548x over previous
import functools

import jax
import jax.numpy as jnp
from jax import lax
from jax.experimental import pallas as pl
from jax.experimental.pallas import tpu as pltpu
from jax.experimental.pallas import tpu_sc as plsc

B = 16
TOTAL = 32768
D = 768
DP = 512
LANE = 16
NCOL = D // LANE

NC = 2
NS = 16
NW = NC * NS
SC_ROWS = 14336
TC_ROWS = TOTAL - SC_ROWS
RPW = SC_ROWS // NW
CH = 64
K = RPW // CH
BLK = 1024


def _sc_body(flat_hbm, cu_hbm, out_hbm, buf, cu_v, acc_v, sem):
    cid = lax.axis_index("c")
    sid = lax.axis_index("s")
    w = cid * NS + sid
    base = w * RPW

    def dma(j, pb):
        return pltpu.make_async_copy(
            flat_hbm.at[pl.ds(base + j * CH, CH)], buf.at[pb], sem.at[pb])

    dma(0, 0).start()

    pltpu.sync_copy(cu_hbm.at[pl.ds(0, LANE)], cu_v)
    cu16 = cu_v[...]
    cvals = [cu16[s] for s in range(B)]
    starts = cvals
    ends = cvals[1:] + [jnp.int32(TOTAL)]

    zero = jnp.zeros((LANE,), jnp.float32)
    for r in range(B):
        for c in range(NCOL):
            acc_v[r, pl.ds(c * LANE, LANE)] = zero

    def chunk_step(j, _):
        cbase = base + j * CH
        pb = lax.rem(j, 2)

        @pl.when(j + 1 < K)
        def _():
            dma(j + 1, 1 - pb).start()

        dma(j, pb).wait()

        def row_step_range(lo, hi):
            def row_step(r, accs, pb=pb):
                return tuple(a + buf[pb, r, pl.ds(c * LANE, LANE)]
                             for c, a in enumerate(accs))
            return lax.fori_loop(lo, hi, row_step,
                                 tuple(zero for _ in range(NCOL)))

        one = jnp.int32(1)
        nil = jnp.int32(0)
        segf = nil
        segl = nil
        for s in range(B):
            segf = segf + jnp.where(ends[s] <= cbase, one, nil)
            segl = segl + jnp.where(ends[s] <= cbase + (CH - 1), one, nil)
        crosses = segf != segl

        @pl.when(jnp.logical_not(crosses))
        def _fast():
            accs = row_step_range(0, CH)
            for c in range(NCOL):
                plsc.addupdate(acc_v.at[segf, pl.ds(c * LANE, LANE)],
                               accs[c])

        @pl.when(crosses)
        def _slow():
            for s in range(B):
                lo = jnp.clip(starts[s] - cbase, 0, CH)
                hi = jnp.clip(ends[s] - cbase, 0, CH)
                accs = row_step_range(lo, hi)

                @pl.when(hi > lo)
                def _(s=s, accs=accs):
                    for c in range(NCOL):
                        plsc.addupdate(
                            acc_v.at[s, pl.ds(c * LANE, LANE)],
                            accs[c])
        return 0

    lax.fori_loop(0, K, chunk_step, 0)

    pltpu.sync_copy(acc_v, out_hbm.at[cid, sid])


@functools.cache
def _sc_segsum():
    mesh = plsc.VectorSubcoreMesh(core_axis_name="c", subcore_axis_name="s")
    return pl.kernel(
        _sc_body,
        mesh=mesh,
        out_type=jax.ShapeDtypeStruct((NC, NS, B, D), jnp.float32),
        scratch_types=[
            pltpu.VMEM((2, CH, D), jnp.float32),
            pltpu.VMEM((LANE,), jnp.int32),
            pltpu.VMEM((B, D), jnp.float32),
            pltpu.SemaphoreType.DMA((2,)),
        ],
    )


def _tc_segsum_body(flat_ref, cu_lo_ref, cu_hi_ref, out_ref):
    i = pl.program_id(0)
    rows = (SC_ROWS + i * BLK
            + lax.broadcasted_iota(jnp.int32, (1, BLK), 1))
    oh = ((rows >= cu_lo_ref[...]) & (rows < cu_hi_ref[...]))
    partial = jnp.dot(oh.astype(jnp.bfloat16),
                      flat_ref[...].astype(jnp.bfloat16),
                      preferred_element_type=jnp.float32)

    @pl.when(i == 0)
    def _():
        out_ref[...] = partial

    @pl.when(i != 0)
    def _():
        out_ref[...] = out_ref[...] + partial


_tc_segsum = pl.pallas_call(
    _tc_segsum_body,
    grid=(TC_ROWS // BLK,),
    in_specs=[
        pl.BlockSpec((BLK, D), lambda i: (SC_ROWS // BLK + i, 0)),
        pl.BlockSpec((B, 1), lambda i: (0, 0)),
        pl.BlockSpec((B, 1), lambda i: (0, 0)),
    ],
    out_specs=pl.BlockSpec((B, D), lambda i: (0, 0)),
    out_shape=jax.ShapeDtypeStruct((B, D), jnp.float32),
)


def _finish_body(part_ref, tcpart_ref, cu_lo_ref, cu_hi_ref, proj_ref,
                 out_ref):
    part = part_ref[...]
    pooled = jnp.sum(part.reshape(NC * NS, B, D), axis=0) + tcpart_ref[...]
    cnt = (cu_hi_ref[...] - cu_lo_ref[...]).astype(jnp.float32)
    pooled = pooled / jnp.maximum(cnt, 1.0)
    ce = jnp.dot(pooled, proj_ref[...], preferred_element_type=jnp.float32)
    n = jnp.sqrt(jnp.sum(ce * ce, axis=1, keepdims=True))
    out_ref[...] = ce / (n + 1e-7)


_finish = pl.pallas_call(
    _finish_body,
    out_shape=jax.ShapeDtypeStruct((B, DP), jnp.float32),
)


def kernel(flat, cu_seqlens, lang_proj):
    cu = cu_seqlens.astype(jnp.int32)
    partial = _sc_segsum()(flat, cu)
    cu_lo = cu[:-1].reshape(B, 1)
    cu_hi = cu[1:].reshape(B, 1)
    tcpart = _tc_segsum(flat, cu_lo, cu_hi)
    return _finish(partial, tcpart, cu_lo, cu_hi, lang_proj)


# speedup vs baseline: 6.7099x; 1.0176x over previous
import functools

import jax
import jax.numpy as jnp
from jax import lax
from jax.experimental import pallas as pl
from jax.experimental.pallas import tpu as pltpu
from jax.experimental.pallas import tpu_sc as plsc

B = 16
TOTAL = 32768
D = 768
DP = 512
LANE = 16
NCOL = D // LANE

NC = 2
NS = 16
NW = NC * NS
SC_ROWS = 12288
TC_ROWS = TOTAL - SC_ROWS
RPW = SC_ROWS // NW
CH = 64
K = RPW // CH
BLK = 1024


def _sc_body(flat_hbm, cu_hbm, out_hbm, buf, cu_v, acc_v, sem):
    cid = lax.axis_index("c")
    sid = lax.axis_index("s")
    w = cid * NS + sid
    base = w * RPW

    def dma(j, pb):
        return pltpu.make_async_copy(
            flat_hbm.at[pl.ds(base + j * CH, CH)], buf.at[pb], sem.at[pb])

    dma(0, 0).start()

    pltpu.sync_copy(cu_hbm.at[pl.ds(0, LANE)], cu_v)
    cu16 = cu_v[...]
    cvals = [cu16[s] for s in range(B)]
    starts = cvals
    ends = cvals[1:] + [jnp.int32(TOTAL)]

    zero = jnp.zeros((LANE,), jnp.float32)
    for r in range(B):
        for c in range(NCOL):
            acc_v[r, pl.ds(c * LANE, LANE)] = zero

    def chunk_step(j, _):
        cbase = base + j * CH
        pb = lax.rem(j, 2)

        @pl.when(j + 1 < K)
        def _():
            dma(j + 1, 1 - pb).start()

        dma(j, pb).wait()

        def row_step_range(lo, hi):
            def row_step(r, accs, pb=pb):
                return tuple(a + buf[pb, r, pl.ds(c * LANE, LANE)]
                             for c, a in enumerate(accs))
            return lax.fori_loop(lo, hi, row_step,
                                 tuple(zero for _ in range(NCOL)))

        one = jnp.int32(1)
        nil = jnp.int32(0)
        segf = nil
        segl = nil
        for s in range(B):
            segf = segf + jnp.where(ends[s] <= cbase, one, nil)
            segl = segl + jnp.where(ends[s] <= cbase + (CH - 1), one, nil)
        crosses = segf != segl

        @pl.when(jnp.logical_not(crosses))
        def _fast():
            accs = row_step_range(0, CH)
            for c in range(NCOL):
                plsc.addupdate(acc_v.at[segf, pl.ds(c * LANE, LANE)],
                               accs[c])

        @pl.when(crosses)
        def _slow():
            for s in range(B):
                lo = jnp.clip(starts[s] - cbase, 0, CH)
                hi = jnp.clip(ends[s] - cbase, 0, CH)
                accs = row_step_range(lo, hi)

                @pl.when(hi > lo)
                def _(s=s, accs=accs):
                    for c in range(NCOL):
                        plsc.addupdate(
                            acc_v.at[s, pl.ds(c * LANE, LANE)],
                            accs[c])
        return 0

    lax.fori_loop(0, K, chunk_step, 0)

    pltpu.sync_copy(acc_v, out_hbm.at[cid, sid])


@functools.cache
def _sc_segsum():
    mesh = plsc.VectorSubcoreMesh(core_axis_name="c", subcore_axis_name="s")
    return pl.kernel(
        _sc_body,
        mesh=mesh,
        out_type=jax.ShapeDtypeStruct((NC, NS, B, D), jnp.float32),
        scratch_types=[
            pltpu.VMEM((2, CH, D), jnp.float32),
            pltpu.VMEM((LANE,), jnp.int32),
            pltpu.VMEM((B, D), jnp.float32),
            pltpu.SemaphoreType.DMA((2,)),
        ],
    )


def _tc_segsum_body(flat_ref, cu_lo_ref, cu_hi_ref, out_ref):
    i = pl.program_id(0)
    rows = (SC_ROWS + i * BLK
            + lax.broadcasted_iota(jnp.int32, (1, BLK), 1))
    oh = ((rows >= cu_lo_ref[...]) & (rows < cu_hi_ref[...]))
    partial = jnp.dot(oh.astype(jnp.bfloat16),
                      flat_ref[...].astype(jnp.bfloat16),
                      preferred_element_type=jnp.float32)

    @pl.when(i == 0)
    def _():
        out_ref[...] = partial

    @pl.when(i != 0)
    def _():
        out_ref[...] = out_ref[...] + partial


_tc_segsum = pl.pallas_call(
    _tc_segsum_body,
    grid=(TC_ROWS // BLK,),
    in_specs=[
        pl.BlockSpec((BLK, D), lambda i: (SC_ROWS // BLK + i, 0)),
        pl.BlockSpec((B, 1), lambda i: (0, 0)),
        pl.BlockSpec((B, 1), lambda i: (0, 0)),
    ],
    out_specs=pl.BlockSpec((B, D), lambda i: (0, 0)),
    out_shape=jax.ShapeDtypeStruct((B, D), jnp.float32),
)


def _finish_body(part_ref, tcpart_ref, cu_lo_ref, cu_hi_ref, proj_ref,
                 out_ref):
    part = part_ref[...]
    pooled = jnp.sum(part.reshape(NC * NS, B, D), axis=0) + tcpart_ref[...]
    cnt = (cu_hi_ref[...] - cu_lo_ref[...]).astype(jnp.float32)
    pooled = pooled / jnp.maximum(cnt, 1.0)
    ce = jnp.dot(pooled, proj_ref[...], preferred_element_type=jnp.float32)
    n = jnp.sqrt(jnp.sum(ce * ce, axis=1, keepdims=True))
    out_ref[...] = ce / (n + 1e-7)


_finish = pl.pallas_call(
    _finish_body,
    out_shape=jax.ShapeDtypeStruct((B, DP), jnp.float32),
)


def kernel(flat, cu_seqlens, lang_proj):
    cu = cu_seqlens.astype(jnp.int32)
    partial = _sc_segsum()(flat, cu)
    cu_lo = cu[:-1].reshape(B, 1)
    cu_hi = cu[1:].reshape(B, 1)
    tcpart = _tc_segsum(flat, cu_lo, cu_hi)
    return _finish(partial, tcpart, cu_lo, cu_hi, lang_proj)
